# Initial kernel scaffold; baseline (speedup 1.0000x reference)
import jax
import jax.numpy as jnp
from jax.experimental import pallas as pl


def kernel(obs, edge_index, W1, b1, W2, b2, W3, b3, Wf, bf):
    raise NotImplementedError("write your pallas kernel here")


# speedup vs baseline: 11.5254x; 11.5254x over previous
import jax
import jax.numpy as jnp
from jax.experimental import pallas as pl
from jax.experimental.pallas import tpu as pltpu

_N = 16
_E = 36


def _elu(x):
    return jnp.where(x > 0, x, jnp.exp(jnp.minimum(x, 0.0)) - 1.0)


def _fused_gcn(ei_ref, obs_ref, w1_ref, b1_ref, w2_ref, b2_ref,
               w3_ref, b3_ref, wf_ref, bf_ref, out_ref):
    si = ei_ref[0:1, :]
    di = ei_ref[1:2, :]

    node = jax.lax.broadcasted_iota(jnp.int32, (_N, _E), 0)
    s_oh = (node == si).astype(jnp.float32)
    d_oh = (node == di).astype(jnp.float32)

    deg = jnp.sum(d_oh, axis=1, keepdims=True) + 1.0
    dis = jax.lax.rsqrt(deg)

    dis_s = jnp.sum(s_oh * dis, axis=0, keepdims=True)
    dis_d = jnp.sum(d_oh * dis, axis=0, keepdims=True)
    norm = dis_s * dis_d

    a_hat = jnp.dot(d_oh * norm, s_oh.T, preferred_element_type=jnp.float32,
                    precision=jax.lax.Precision.HIGHEST)
    ii = jax.lax.broadcasted_iota(jnp.int32, (_N, _N), 0)
    jj = jax.lax.broadcasted_iota(jnp.int32, (_N, _N), 1)
    eye = (ii == jj).astype(jnp.float32)
    a_hat = a_hat + eye * (dis * dis)

    def layer(x, w_ref, b_ref):
        h = jnp.dot(x, w_ref[...], preferred_element_type=jnp.float32,
                    precision=jax.lax.Precision.HIGHEST)
        agg = jnp.dot(a_hat, h, preferred_element_type=jnp.float32,
                    precision=jax.lax.Precision.HIGHEST)
        return _elu(agg + b_ref[...])

    x = layer(obs_ref[...], w1_ref, b1_ref)
    x = layer(x, w2_ref, b2_ref)
    x = layer(x, w3_ref, b3_ref)
    pooled = jnp.max(x, axis=0, keepdims=True)
    out = jnp.dot(pooled, wf_ref[...], preferred_element_type=jnp.float32,
                    precision=jax.lax.Precision.HIGHEST)
    out_ref[...] = _elu(out + bf_ref[...])


def kernel(obs, edge_index, W1, b1, W2, b2, W3, b3, Wf, bf):
    ei = edge_index.astype(jnp.int32)
    out = pl.pallas_call(
        _fused_gcn,
        out_shape=jax.ShapeDtypeStruct((1, 256), jnp.float32),
    )(ei, obs,
      W1, b1.reshape(1, -1), W2, b2.reshape(1, -1),
      W3, b3.reshape(1, -1), Wf, bf.reshape(1, -1))
    return out.reshape(256)


# speedup vs baseline: 11.5822x; 1.0049x over previous
import jax
import jax.numpy as jnp
from jax.experimental import pallas as pl
from jax.experimental.pallas import tpu as pltpu

_N = 16
_E = 36


def _elu(x):
    return jnp.where(x > 0, x, jnp.exp(jnp.minimum(x, 0.0)) - 1.0)


def _fused_gcn(ei_ref, obs_ref, w1_ref, b1_ref, w2_ref, b2_ref,
               w3_ref, b3_ref, wf_ref, bf_ref, out_ref):
    si = ei_ref[0:1, :]
    di = ei_ref[1:2, :]

    node = jax.lax.broadcasted_iota(jnp.int32, (_N, _E), 0)
    s_oh = (node == si).astype(jnp.float32)
    d_oh = (node == di).astype(jnp.float32)

    deg = jnp.sum(d_oh, axis=1, keepdims=True) + 1.0
    dis = jax.lax.rsqrt(deg)

    dis_s = jnp.sum(s_oh * dis, axis=0, keepdims=True)
    dis_d = jnp.sum(d_oh * dis, axis=0, keepdims=True)
    norm = dis_s * dis_d

    a_hat = jnp.dot(d_oh * norm, s_oh.T, preferred_element_type=jnp.float32,
                    precision=jax.lax.Precision.HIGHEST)
    ii = jax.lax.broadcasted_iota(jnp.int32, (_N, _N), 0)
    jj = jax.lax.broadcasted_iota(jnp.int32, (_N, _N), 1)
    eye = (ii == jj).astype(jnp.float32)
    a_hat = a_hat + eye * (dis * dis)

    def layer(x, w_ref, b_ref):
        h = jnp.dot(x, w_ref[...], preferred_element_type=jnp.float32,
                    precision=jax.lax.Precision.HIGHEST)
        agg = jnp.dot(a_hat, h, preferred_element_type=jnp.float32,
                    precision=jax.lax.Precision.HIGHEST)
        return _elu(agg + b_ref[...])

    x = layer(obs_ref[...], w1_ref, b1_ref)
    x = layer(x, w2_ref, b2_ref)
    x = layer(x, w3_ref, b3_ref)
    pooled = jnp.max(x, axis=0, keepdims=True)
    out = jnp.dot(pooled, wf_ref[...], preferred_element_type=jnp.float32,
                    precision=jax.lax.Precision.HIGHEST)
    out_ref[...] = _elu(out + bf_ref[...]).reshape(256)


def kernel(obs, edge_index, W1, b1, W2, b2, W3, b3, Wf, bf):
    out = pl.pallas_call(
        _fused_gcn,
        out_shape=jax.ShapeDtypeStruct((256,), jnp.float32),
    )(edge_index.astype(jnp.int32), obs, W1, b1, W2, b2, W3, b3, Wf, bf)
    return out
